# baseline (speedup 1.0000x reference)
import functools

import jax
import jax.numpy as jnp
from jax import lax
from jax.experimental import pallas as pl
from jax.experimental.pallas import tpu as pltpu
from jax.experimental.pallas import tpu_sc as plsc

_DELTA = 5.0
_R = 0.01
_NEG_FILL = -1e9

_M, _N = 1024, 16384
_K = int(_R * (_N - 1))
_NW = 32
_SROWS = 640
_ROWS_PER = _SROWS // _NW
_NVEC = _N // 16
_U = 8
_HPAD = 4096
_TROWS = 16
_TGTBUF = 32


def _scal(v):
    return lax.reduce_max(v, axes=(0,)) if getattr(v, "ndim", 0) else v


def _sscal(v):
    return lax.reduce_sum(v, axes=(0,)) if getattr(v, "ndim", 0) else v


def _key_vec(xv):
    iv = plsc.bitcast(xv, jnp.int32)
    return iv ^ (lax.shift_right_arithmetic(iv, 31) & jnp.int32(0x7FFFFFFF))


def _float_of_key_vec(kscal):
    kv = jnp.full((16,), kscal, dtype=jnp.int32)
    iv = jnp.where(kv >= 0, kv, kv ^ jnp.int32(0x7FFFFFFF))
    return plsc.bitcast(iv, jnp.float32)


def _scan_histogram(h_ref, k_rem):
    iota16 = lax.iota(jnp.int32, 16)
    tsum = []
    for c in range(16):
        acc = jnp.zeros((16,), jnp.float32)
        for d in range(16):
            acc = acc + h_ref[pl.ds((c * 16 + d) * 16, 16)]
        tsum.append(_sscal(acc))
    suf = [jnp.float32(0.0)] * 17
    for c in range(15, -1, -1):
        suf[c] = suf[c + 1] + tsum[c]
    suf_v = jnp.zeros((16,), jnp.float32)
    for c in range(16):
        suf_v = jnp.where(iota16 == c, suf[c], suf_v)
    m1 = suf_v >= k_rem
    c_star = _scal(plsc.all_reduce_population_count(m1)) - 1
    above_v = jnp.zeros((16,), jnp.float32)
    for c in range(16):
        above_v = jnp.where(jnp.int32(c) == c_star + 1, suf[c], above_v)
    above = _scal(above_v)
    tc = jnp.zeros((16,), jnp.float32)
    for j in range(16):
        bv = h_ref[pl.ds((c_star * 16 + j) * 16, 16)]
        tc = jnp.where(iota16 == j, _sscal(bv), tc)
    total_c = _sscal(tc)
    ws = total_c - plsc.cumsum(tc) + tc
    m2 = (above + ws) >= k_rem
    j_star = _scal(plsc.all_reduce_population_count(m2)) - 1
    ws_j = _sscal(jnp.where(iota16 == j_star, ws, 0.0))
    cnt_j = _sscal(jnp.where(iota16 == j_star, tc, 0.0))
    bucket = c_star * 16 + j_star
    c_above = above + (ws_j - cnt_j)
    return bucket, c_above, cnt_j


def _sc_body(inputs_hbm, targets_hbm, out_hbm, rowbuf, h1, h2, tgtv, outv,
             sem0, sem1):
    nc = 2
    wid = lax.axis_index("s") * nc + lax.axis_index("c")
    base = wid * _ROWS_PER
    iota16 = lax.iota(jnp.int32, 16)
    ones = jnp.full((16,), 1.0, dtype=jnp.float32)
    lane0 = iota16 == 0
    kf = jnp.float32(_K)

    toff = base & 7
    abase = pl.multiple_of(base - toff, 8)
    pltpu.sync_copy(targets_hbm.at[pl.ds(abase, _TGTBUF)], tgtv)

    def process_row(b, r, ls):
        bvec = jnp.full((16,), b, jnp.int32)
        tvec = plsc.load_gather(tgtv, [jnp.full((16,), toff + r, jnp.int32)])
        pos16 = plsc.load_gather(rowbuf, [bvec, tvec])
        pos = _scal(pos16)
        plsc.store_scatter(rowbuf, [bvec, tvec],
                           jnp.full((16,), _NEG_FILL, jnp.float32), mask=lane0)

        def zero_body(i, _):
            for u in range(_U):
                h1[pl.ds((i * _U + u) * 16, 16)] = jnp.zeros((16,), jnp.float32)
                h2[pl.ds((i * _U + u) * 16, 16)] = jnp.zeros((16,), jnp.float32)
            return 0

        lax.fori_loop(0, _HPAD // 16 // _U, zero_body, 0)

        def l1_body(i, _):
            xs = [rowbuf[b, pl.ds((i * _U + u) * 16, 16)] for u in range(_U)]
            addrs = []
            for u in range(_U):
                key = _key_vec(xs[u])
                d1 = lax.shift_right_arithmetic(key, 24) + 128
                addrs.append(lax.shift_left(d1, 4) + iota16)
            for u in range(_U):
                plsc.addupdate_scatter(h1, [addrs[u]], ones)
            return 0

        lax.fori_loop(0, _NVEC // _U, l1_body, 0)
        b1, c_gt1, _ = _scan_histogram(h1, kf)

        def l2_body(i, _):
            xs = [rowbuf[b, pl.ds((i * _U + u) * 16, 16)] for u in range(_U)]
            addrs, masks = [], []
            for u in range(_U):
                key = _key_vec(xs[u])
                d1 = lax.shift_right_arithmetic(key, 24) + 128
                d2s = lax.shift_right_arithmetic(key, 12) & jnp.int32(0xFF0)
                addrs.append(d2s + iota16)
                masks.append(d1 == b1)
            for u in range(_U):
                plsc.addupdate_scatter(h2, [addrs[u]], ones, mask=masks[u])
            return 0

        lax.fori_loop(0, _NVEC // _U, l2_body, 0)
        b2, c_gt2, c_b = _scan_histogram(h2, kf - c_gt1)
        c_gt = c_gt1 + c_gt2

        lo_key = lax.shift_left(b1 - 128, 24) + lax.shift_left(b2, 16)
        hi_key = lo_key + jnp.int32(1 << 16)
        lo_fv = _float_of_key_vec(lo_key)
        hi_fv = _float_of_key_vec(hi_key)

        def st_body(i, carry):
            ga, gb, ea, eb = carry
            xs = [rowbuf[b, pl.ds((i * _U + u) * 16, 16)] for u in range(_U)]
            sqs = [(1.0 + xv) * (1.0 + xv) for xv in xs]
            gts = [jnp.where(xv >= hi_fv, sq, 0.0) for xv, sq in zip(xs, sqs)]
            ges = [jnp.where(xv >= lo_fv, sq, 0.0) for xv, sq in zip(xs, sqs)]
            for u in range(0, _U, 2):
                ga = ga + gts[u]
                ea = ea + ges[u]
                gb = gb + gts[u + 1]
                eb = eb + ges[u + 1]
            return ga, gb, ea, eb

        z16 = jnp.zeros((16,), jnp.float32)
        ga, gb, ea, eb = lax.fori_loop(0, _NVEC // _U, st_body,
                                       (z16, z16, z16, z16))
        s_gt = _sscal(ga + gb)
        s_b = _sscal(ea + eb) - s_gt
        ratio = _scal(jnp.full((16,), s_b) / jnp.full((16,), c_b))
        neg = (s_gt + (kf - c_gt) * ratio) * jnp.float32(1.0 / _K)
        l_r = _DELTA * (1.0 - pos) * (1.0 - pos) + neg
        return ls + l_r

    pltpu.make_async_copy(inputs_hbm.at[base], rowbuf.at[0], sem0).start()

    def pair_body(t, ls):
        r0 = 2 * t
        r1 = r0 + 1
        pltpu.make_async_copy(inputs_hbm.at[base + r0], rowbuf.at[0], sem0).wait()
        pltpu.make_async_copy(inputs_hbm.at[base + r1], rowbuf.at[1], sem1).start()
        ls = process_row(0, r0, ls)
        pltpu.make_async_copy(inputs_hbm.at[base + r1], rowbuf.at[1], sem1).wait()
        nxt = jnp.minimum(base + r0 + 2, jnp.int32(_M - 1))
        pltpu.make_async_copy(inputs_hbm.at[nxt], rowbuf.at[0], sem0).start()
        ls = process_row(1, r1, ls)
        return ls

    ls = lax.fori_loop(0, _ROWS_PER // 2, pair_body, jnp.float32(0.0))
    pltpu.make_async_copy(inputs_hbm.at[jnp.minimum(base + _ROWS_PER,
                                                    jnp.int32(_M - 1))],
                          rowbuf.at[0], sem0).wait()
    outv[...] = jnp.where(lane0, ls, 0.0)
    pltpu.sync_copy(outv, out_hbm.at[wid])


def _key_of_tc(x):
    i = jax.lax.bitcast_convert_type(x, jnp.int32)
    return i ^ (lax.shift_right_arithmetic(i, 31) & jnp.int32(0x7FFFFFFF))


def _float_of_key_tc(key):
    i = jnp.where(key >= 0, key, key ^ jnp.int32(0x7FFFFFFF))
    return jax.lax.bitcast_convert_type(i, jnp.float32)


def _tc_block(inputs_ref, tgt_ref, out_ref, *, k, n, rows):
    x = inputs_ref[...]
    tgt = tgt_ref[...]
    col = jax.lax.broadcasted_iota(jnp.int32, (rows, n), 1)
    pos_mask = col == tgt
    pos = jnp.sum(jnp.where(pos_mask, x, 0.0), axis=1, keepdims=True)
    x = jnp.where(pos_mask, jnp.float32(_NEG_FILL), x)

    kf = jnp.float32(k)
    mx = jnp.max(x, axis=1, keepdims=True)
    mn = jnp.min(x, axis=1, keepdims=True)
    lo = _key_of_tc(mn)
    hi = _key_of_tc(mx) + 1

    def body(_, carry):
        lo, hi = carry
        mid = (lo & hi) + lax.shift_right_arithmetic(lo ^ hi, 1)
        t = _float_of_key_tc(mid)
        cnt = jnp.sum(jnp.where(x >= t, 1.0, 0.0), axis=1, keepdims=True)
        take = cnt >= kf
        return jnp.where(take, mid, lo), jnp.where(take, hi, mid)

    lo, hi = jax.lax.fori_loop(0, 18, body, (lo, hi))
    lo_f = _float_of_key_tc(lo)
    hi_f = _float_of_key_tc(hi)
    sq = (1.0 + x) ** 2
    ge_hi = x >= hi_f
    ge_lo = x >= lo_f
    cnt_gt = jnp.sum(jnp.where(ge_hi, 1.0, 0.0), axis=1, keepdims=True)
    s_gt = jnp.sum(jnp.where(ge_hi, sq, 0.0), axis=1, keepdims=True)
    cnt_ge = jnp.sum(jnp.where(ge_lo, 1.0, 0.0), axis=1, keepdims=True)
    s_ge = jnp.sum(jnp.where(ge_lo, sq, 0.0), axis=1, keepdims=True)
    cnt_b = cnt_ge - cnt_gt
    s_b = s_ge - s_gt
    neg = (s_gt + (kf - cnt_gt) * s_b / cnt_b) / kf
    l = _DELTA * (1.0 - pos) ** 2 + neg

    @pl.when(pl.program_id(0) == 0)
    def _():
        out_ref[...] = jnp.zeros_like(out_ref)

    out_ref[...] += jnp.sum(l, axis=0, keepdims=True)


def kernel(inputs, targets_, targets, GT_MC):
    m, n = inputs.shape
    tgt_i32 = targets.astype(jnp.int32)
    mesh = plsc.VectorSubcoreMesh(core_axis_name="c", subcore_axis_name="s")
    sc_fn = functools.partial(
        pl.kernel,
        mesh=mesh,
        compiler_params=pltpu.CompilerParams(needs_layout_passes=False),
        out_type=jax.ShapeDtypeStruct((_NW, 16), jnp.float32),
        scratch_types=[
            pltpu.VMEM((2, _N), jnp.float32),
            pltpu.VMEM((_HPAD,), jnp.float32),
            pltpu.VMEM((_HPAD,), jnp.float32),
            pltpu.VMEM((_TGTBUF,), jnp.int32),
            pltpu.VMEM((16,), jnp.float32),
            pltpu.SemaphoreType.DMA,
            pltpu.SemaphoreType.DMA,
        ],
    )(_sc_body)
    sc_out = sc_fn(inputs, tgt_i32)

    tc_grid = (m - _SROWS) // _TROWS
    off = _SROWS // _TROWS
    tc_out = pl.pallas_call(
        functools.partial(_tc_block, k=_K, n=n, rows=_TROWS),
        grid=(tc_grid,),
        in_specs=[
            pl.BlockSpec((_TROWS, n), lambda i: (i + off, 0)),
            pl.BlockSpec((_TROWS, 1), lambda i: (i + off, 0)),
        ],
        out_specs=pl.BlockSpec((1, 1), lambda i: (0, 0)),
        out_shape=jax.ShapeDtypeStruct((1, 1), jnp.float32),
    )(inputs, tgt_i32.reshape(m, 1))

    return (jnp.sum(sc_out) + tc_out[0, 0]) / m


# speedup vs baseline: 1.0282x; 1.0282x over previous
import functools

import jax
import jax.numpy as jnp
from jax import lax
from jax.experimental import pallas as pl
from jax.experimental.pallas import tpu as pltpu
from jax.experimental.pallas import tpu_sc as plsc

_DELTA = 5.0
_R = 0.01
_NEG_FILL = -1e9

_M, _N = 1024, 16384
_K = int(_R * (_N - 1))
_NW = 32
_SROWS = 576
_ROWS_PER = _SROWS // _NW
_NVEC = _N // 16
_U = 8
_HPAD = 4096
_TROWS = 16
_TGTBUF = 32


def _scal(v):
    return lax.reduce_max(v, axes=(0,)) if getattr(v, "ndim", 0) else v


def _sscal(v):
    return lax.reduce_sum(v, axes=(0,)) if getattr(v, "ndim", 0) else v


def _key_vec(xv):
    iv = plsc.bitcast(xv, jnp.int32)
    return iv ^ (lax.shift_right_arithmetic(iv, 31) & jnp.int32(0x7FFFFFFF))


def _float_of_key_vec(kscal):
    kv = jnp.full((16,), kscal, dtype=jnp.int32)
    iv = jnp.where(kv >= 0, kv, kv ^ jnp.int32(0x7FFFFFFF))
    return plsc.bitcast(iv, jnp.float32)


def _scan_histogram(h_ref, q_ref, k_rem):
    iota16 = lax.iota(jnp.int32, 16)
    tsum, qsum = [], []
    for c in range(16):
        acc = jnp.zeros((16,), jnp.float32)
        qacc = jnp.zeros((16,), jnp.float32)
        for d in range(16):
            acc = acc + h_ref[pl.ds((c * 16 + d) * 16, 16)]
            qacc = qacc + q_ref[pl.ds((c * 16 + d) * 16, 16)]
        tsum.append(_sscal(acc))
        qsum.append(_sscal(qacc))
    suf = [jnp.float32(0.0)] * 17
    qsuf = [jnp.float32(0.0)] * 17
    for c in range(15, -1, -1):
        suf[c] = suf[c + 1] + tsum[c]
        qsuf[c] = qsuf[c + 1] + qsum[c]
    suf_v = jnp.zeros((16,), jnp.float32)
    for c in range(16):
        suf_v = jnp.where(iota16 == c, suf[c], suf_v)
    m1 = suf_v >= k_rem
    c_star = _scal(plsc.all_reduce_population_count(m1)) - 1
    above_v = jnp.zeros((16,), jnp.float32)
    qabove_v = jnp.zeros((16,), jnp.float32)
    for c in range(16):
        sel = jnp.int32(c) == c_star + 1
        above_v = jnp.where(sel, suf[c], above_v)
        qabove_v = jnp.where(sel, qsuf[c], qabove_v)
    above = _scal(above_v)
    qabove = _scal(qabove_v)
    tc = jnp.zeros((16,), jnp.float32)
    qc = jnp.zeros((16,), jnp.float32)
    for j in range(16):
        bv = h_ref[pl.ds((c_star * 16 + j) * 16, 16)]
        qv = q_ref[pl.ds((c_star * 16 + j) * 16, 16)]
        tc = jnp.where(iota16 == j, _sscal(bv), tc)
        qc = jnp.where(iota16 == j, _sscal(qv), qc)
    total_c = _sscal(tc)
    ws = total_c - plsc.cumsum(tc) + tc
    qws = _sscal(qc) - plsc.cumsum(qc) + qc
    m2 = (above + ws) >= k_rem
    j_star = _scal(plsc.all_reduce_population_count(m2)) - 1
    ws_j = _sscal(jnp.where(iota16 == j_star, ws, 0.0))
    cnt_j = _sscal(jnp.where(iota16 == j_star, tc, 0.0))
    qws_j = _sscal(jnp.where(iota16 == j_star, qws, 0.0))
    sq_j = _sscal(jnp.where(iota16 == j_star, qc, 0.0))
    bucket = c_star * 16 + j_star
    c_above = above + (ws_j - cnt_j)
    sq_above = qabove + (qws_j - sq_j)
    return bucket, c_above, cnt_j, sq_above, sq_j


def _sc_body(inputs_hbm, targets_hbm, out_hbm, rowbuf, h1, h2, q1, q2, tgtv,
             outv, sem0, sem1):
    nc = 2
    wid = lax.axis_index("s") * nc + lax.axis_index("c")
    base = wid * _ROWS_PER
    iota16 = lax.iota(jnp.int32, 16)
    ones = jnp.full((16,), 1.0, dtype=jnp.float32)
    lane0 = iota16 == 0
    kf = jnp.float32(_K)

    toff = base & 7
    abase = pl.multiple_of(base - toff, 8)
    pltpu.sync_copy(targets_hbm.at[pl.ds(abase, _TGTBUF)], tgtv)

    def process_row(b, r, ls):
        bvec = jnp.full((16,), b, jnp.int32)
        tvec = plsc.load_gather(tgtv, [jnp.full((16,), toff + r, jnp.int32)])
        pos16 = plsc.load_gather(rowbuf, [bvec, tvec])
        pos = _scal(pos16)
        plsc.store_scatter(rowbuf, [bvec, tvec],
                           jnp.full((16,), _NEG_FILL, jnp.float32), mask=lane0)

        def zero_body(i, _):
            z = jnp.zeros((16,), jnp.float32)
            for u in range(_U):
                h1[pl.ds((i * _U + u) * 16, 16)] = z
                h2[pl.ds((i * _U + u) * 16, 16)] = z
                q1[pl.ds((i * _U + u) * 16, 16)] = z
                q2[pl.ds((i * _U + u) * 16, 16)] = z
            return 0

        lax.fori_loop(0, _HPAD // 16 // _U, zero_body, 0)

        def l1_body(i, _):
            xs = [rowbuf[b, pl.ds((i * _U + u) * 16, 16)] for u in range(_U)]
            addrs, sqs = [], []
            for u in range(_U):
                key = _key_vec(xs[u])
                d1 = lax.shift_right_arithmetic(key, 24) + 128
                addrs.append(lax.shift_left(d1, 4) + iota16)
                sqs.append((1.0 + xs[u]) * (1.0 + xs[u]))
            for u in range(_U):
                plsc.addupdate_scatter(h1, [addrs[u]], ones)
                plsc.addupdate_scatter(q1, [addrs[u]], sqs[u])
            return 0

        lax.fori_loop(0, _NVEC // _U, l1_body, 0)
        b1, c_gt1, _, s_gt1, _ = _scan_histogram(h1, q1, kf)

        def l2_body(i, _):
            xs = [rowbuf[b, pl.ds((i * _U + u) * 16, 16)] for u in range(_U)]
            addrs, masks, sqs = [], [], []
            for u in range(_U):
                key = _key_vec(xs[u])
                d1 = lax.shift_right_arithmetic(key, 24) + 128
                d2s = lax.shift_right_arithmetic(key, 12) & jnp.int32(0xFF0)
                addrs.append(d2s + iota16)
                masks.append(d1 == b1)
                sqs.append((1.0 + xs[u]) * (1.0 + xs[u]))
            for u in range(_U):
                plsc.addupdate_scatter(h2, [addrs[u]], ones, mask=masks[u])
                plsc.addupdate_scatter(q2, [addrs[u]], sqs[u], mask=masks[u])
            return 0

        lax.fori_loop(0, _NVEC // _U, l2_body, 0)
        b2, c_gt2, c_b, s_gt2, s_b = _scan_histogram(h2, q2, kf - c_gt1)
        c_gt = c_gt1 + c_gt2
        s_gt = s_gt1 + s_gt2
        ratio = _scal(jnp.full((16,), s_b) / jnp.full((16,), c_b))
        neg = (s_gt + (kf - c_gt) * ratio) * jnp.float32(1.0 / _K)
        l_r = _DELTA * (1.0 - pos) * (1.0 - pos) + neg
        return ls + l_r

    pltpu.make_async_copy(inputs_hbm.at[base], rowbuf.at[0], sem0).start()

    def pair_body(t, ls):
        r0 = 2 * t
        r1 = r0 + 1
        pltpu.make_async_copy(inputs_hbm.at[base + r0], rowbuf.at[0], sem0).wait()
        pltpu.make_async_copy(inputs_hbm.at[base + r1], rowbuf.at[1], sem1).start()
        ls = process_row(0, r0, ls)
        pltpu.make_async_copy(inputs_hbm.at[base + r1], rowbuf.at[1], sem1).wait()
        nxt = jnp.minimum(base + r0 + 2, jnp.int32(_M - 1))
        pltpu.make_async_copy(inputs_hbm.at[nxt], rowbuf.at[0], sem0).start()
        ls = process_row(1, r1, ls)
        return ls

    ls = lax.fori_loop(0, _ROWS_PER // 2, pair_body, jnp.float32(0.0))
    pltpu.make_async_copy(inputs_hbm.at[jnp.minimum(base + _ROWS_PER,
                                                    jnp.int32(_M - 1))],
                          rowbuf.at[0], sem0).wait()
    outv[...] = jnp.where(lane0, ls, 0.0)
    pltpu.sync_copy(outv, out_hbm.at[wid])


def _key_of_tc(x):
    i = jax.lax.bitcast_convert_type(x, jnp.int32)
    return i ^ (lax.shift_right_arithmetic(i, 31) & jnp.int32(0x7FFFFFFF))


def _float_of_key_tc(key):
    i = jnp.where(key >= 0, key, key ^ jnp.int32(0x7FFFFFFF))
    return jax.lax.bitcast_convert_type(i, jnp.float32)


def _tc_block(inputs_ref, tgt_ref, out_ref, *, k, n, rows):
    x = inputs_ref[...]
    tgt = tgt_ref[...]
    col = jax.lax.broadcasted_iota(jnp.int32, (rows, n), 1)
    pos_mask = col == tgt
    pos = jnp.sum(jnp.where(pos_mask, x, 0.0), axis=1, keepdims=True)
    x = jnp.where(pos_mask, jnp.float32(_NEG_FILL), x)

    kf = jnp.float32(k)
    mx = jnp.max(x, axis=1, keepdims=True)
    mn = jnp.min(x, axis=1, keepdims=True)
    lo = _key_of_tc(mn)
    hi = _key_of_tc(mx) + 1

    def body(_, carry):
        lo, hi = carry
        mid = (lo & hi) + lax.shift_right_arithmetic(lo ^ hi, 1)
        t = _float_of_key_tc(mid)
        cnt = jnp.sum(jnp.where(x >= t, 1.0, 0.0), axis=1, keepdims=True)
        take = cnt >= kf
        return jnp.where(take, mid, lo), jnp.where(take, hi, mid)

    lo, hi = jax.lax.fori_loop(0, 18, body, (lo, hi))
    lo_f = _float_of_key_tc(lo)
    hi_f = _float_of_key_tc(hi)
    sq = (1.0 + x) ** 2
    ge_hi = x >= hi_f
    ge_lo = x >= lo_f
    cnt_gt = jnp.sum(jnp.where(ge_hi, 1.0, 0.0), axis=1, keepdims=True)
    s_gt = jnp.sum(jnp.where(ge_hi, sq, 0.0), axis=1, keepdims=True)
    cnt_ge = jnp.sum(jnp.where(ge_lo, 1.0, 0.0), axis=1, keepdims=True)
    s_ge = jnp.sum(jnp.where(ge_lo, sq, 0.0), axis=1, keepdims=True)
    cnt_b = cnt_ge - cnt_gt
    s_b = s_ge - s_gt
    neg = (s_gt + (kf - cnt_gt) * s_b / cnt_b) / kf
    l = _DELTA * (1.0 - pos) ** 2 + neg

    @pl.when(pl.program_id(0) == 0)
    def _():
        out_ref[...] = jnp.zeros_like(out_ref)

    out_ref[...] += jnp.sum(l, axis=0, keepdims=True)


def kernel(inputs, targets_, targets, GT_MC):
    m, n = inputs.shape
    tgt_i32 = targets.astype(jnp.int32)
    mesh = plsc.VectorSubcoreMesh(core_axis_name="c", subcore_axis_name="s")
    sc_fn = functools.partial(
        pl.kernel,
        mesh=mesh,
        compiler_params=pltpu.CompilerParams(needs_layout_passes=False),
        out_type=jax.ShapeDtypeStruct((_NW, 16), jnp.float32),
        scratch_types=[
            pltpu.VMEM((2, _N), jnp.float32),
            pltpu.VMEM((_HPAD,), jnp.float32),
            pltpu.VMEM((_HPAD,), jnp.float32),
            pltpu.VMEM((_HPAD,), jnp.float32),
            pltpu.VMEM((_HPAD,), jnp.float32),
            pltpu.VMEM((_TGTBUF,), jnp.int32),
            pltpu.VMEM((16,), jnp.float32),
            pltpu.SemaphoreType.DMA,
            pltpu.SemaphoreType.DMA,
        ],
    )(_sc_body)
    sc_out = sc_fn(inputs, tgt_i32)

    tc_grid = (m - _SROWS) // _TROWS
    off = _SROWS // _TROWS
    tc_out = pl.pallas_call(
        functools.partial(_tc_block, k=_K, n=n, rows=_TROWS),
        grid=(tc_grid,),
        in_specs=[
            pl.BlockSpec((_TROWS, n), lambda i: (i + off, 0)),
            pl.BlockSpec((_TROWS, 1), lambda i: (i + off, 0)),
        ],
        out_specs=pl.BlockSpec((1, 1), lambda i: (0, 0)),
        out_shape=jax.ShapeDtypeStruct((1, 1), jnp.float32),
    )(inputs, tgt_i32.reshape(m, 1))

    return (jnp.sum(sc_out) + tc_out[0, 0]) / m


# speedup vs baseline: 1.0928x; 1.0629x over previous
import functools

import jax
import jax.numpy as jnp
from jax import lax
from jax.experimental import pallas as pl
from jax.experimental.pallas import tpu as pltpu
from jax.experimental.pallas import tpu_sc as plsc

_DELTA = 5.0
_R = 0.01
_NEG_FILL = -1e9

_M, _N = 1024, 16384
_K = int(_R * (_N - 1))
_NW = 32
_SROWS = 576
_ROWS_PER = _SROWS // _NW
_NVEC = _N // 16
_U = 8
_HPAD = 4096
_TROWS = 32
_TGTBUF = 32


def _scal(v):
    return lax.reduce_max(v, axes=(0,)) if getattr(v, "ndim", 0) else v


def _sscal(v):
    return lax.reduce_sum(v, axes=(0,)) if getattr(v, "ndim", 0) else v


def _key_vec(xv):
    iv = plsc.bitcast(xv, jnp.int32)
    return iv ^ (lax.shift_right_arithmetic(iv, 31) & jnp.int32(0x7FFFFFFF))


def _float_of_key_vec(kscal):
    kv = jnp.full((16,), kscal, dtype=jnp.int32)
    iv = jnp.where(kv >= 0, kv, kv ^ jnp.int32(0x7FFFFFFF))
    return plsc.bitcast(iv, jnp.float32)


def _scan_histogram(h_ref, k_rem):
    iota16 = lax.iota(jnp.int32, 16)
    tsum = []
    for c in range(16):
        acc = jnp.zeros((16,), jnp.float32)
        for d in range(16):
            acc = acc + h_ref[pl.ds((c * 16 + d) * 16, 16)]
        tsum.append(_sscal(acc))
    suf = [jnp.float32(0.0)] * 17
    for c in range(15, -1, -1):
        suf[c] = suf[c + 1] + tsum[c]
    suf_v = jnp.zeros((16,), jnp.float32)
    for c in range(16):
        suf_v = jnp.where(iota16 == c, suf[c], suf_v)
    m1 = suf_v >= k_rem
    c_star = _scal(plsc.all_reduce_population_count(m1)) - 1
    above_v = jnp.zeros((16,), jnp.float32)
    for c in range(16):
        above_v = jnp.where(jnp.int32(c) == c_star + 1, suf[c], above_v)
    above = _scal(above_v)
    tc = jnp.zeros((16,), jnp.float32)
    for j in range(16):
        bv = h_ref[pl.ds((c_star * 16 + j) * 16, 16)]
        tc = jnp.where(iota16 == j, _sscal(bv), tc)
    total_c = _sscal(tc)
    ws = total_c - plsc.cumsum(tc) + tc
    m2 = (above + ws) >= k_rem
    j_star = _scal(plsc.all_reduce_population_count(m2)) - 1
    ws_j = _sscal(jnp.where(iota16 == j_star, ws, 0.0))
    cnt_j = _sscal(jnp.where(iota16 == j_star, tc, 0.0))
    bucket = c_star * 16 + j_star
    c_above = above + (ws_j - cnt_j)
    return bucket, c_above, cnt_j


def _sc_body(inputs_hbm, targets_hbm, out_hbm, rowbuf, h1, h2, tgtv, outv,
             sem0, sem1):
    nc = 2
    wid = lax.axis_index("s") * nc + lax.axis_index("c")
    base = wid * _ROWS_PER
    iota16 = lax.iota(jnp.int32, 16)
    ones = jnp.full((16,), 1.0, dtype=jnp.float32)
    lane0 = iota16 == 0
    kf = jnp.float32(_K)

    toff = base & 7
    abase = pl.multiple_of(base - toff, 8)
    pltpu.sync_copy(targets_hbm.at[pl.ds(abase, _TGTBUF)], tgtv)

    def process_row(b, r, ls):
        bvec = jnp.full((16,), b, jnp.int32)
        tvec = plsc.load_gather(tgtv, [jnp.full((16,), toff + r, jnp.int32)])
        pos16 = plsc.load_gather(rowbuf, [bvec, tvec])
        pos = _scal(pos16)
        plsc.store_scatter(rowbuf, [bvec, tvec],
                           jnp.full((16,), _NEG_FILL, jnp.float32), mask=lane0)

        def zero_body(i, _):
            for u in range(_U):
                h1[pl.ds((i * _U + u) * 16, 16)] = jnp.zeros((16,), jnp.float32)
                h2[pl.ds((i * _U + u) * 16, 16)] = jnp.zeros((16,), jnp.float32)
            return 0

        lax.fori_loop(0, _HPAD // 16 // _U, zero_body, 0)

        def l1_body(i, _):
            xs = [rowbuf[b, pl.ds((i * _U + u) * 16, 16)] for u in range(_U)]
            addrs = []
            for u in range(_U):
                key = _key_vec(xs[u])
                d1 = lax.shift_right_arithmetic(key, 24) + 128
                addrs.append(lax.shift_left(d1, 4) + iota16)
            for u in range(_U):
                plsc.addupdate_scatter(h1, [addrs[u]], ones)
            return 0

        lax.fori_loop(0, _NVEC // _U, l1_body, 0)
        b1, c_gt1, _ = _scan_histogram(h1, kf)

        def l2_body(i, _):
            xs = [rowbuf[b, pl.ds((i * _U + u) * 16, 16)] for u in range(_U)]
            addrs, masks = [], []
            for u in range(_U):
                key = _key_vec(xs[u])
                d1 = lax.shift_right_arithmetic(key, 24) + 128
                d2s = lax.shift_right_arithmetic(key, 12) & jnp.int32(0xFF0)
                addrs.append(d2s + iota16)
                masks.append(d1 == b1)
            for u in range(_U):
                plsc.addupdate_scatter(h2, [addrs[u]], ones, mask=masks[u])
            return 0

        lax.fori_loop(0, _NVEC // _U, l2_body, 0)
        b2, c_gt2, c_b = _scan_histogram(h2, kf - c_gt1)
        c_gt = c_gt1 + c_gt2

        lo_key = lax.shift_left(b1 - 128, 24) + lax.shift_left(b2, 16)
        hi_key = lo_key + jnp.int32(1 << 16)
        lo_fv = _float_of_key_vec(lo_key)
        hi_fv = _float_of_key_vec(hi_key)

        def st_body(i, carry):
            ga, gb, ea, eb = carry
            xs = [rowbuf[b, pl.ds((i * _U + u) * 16, 16)] for u in range(_U)]
            sqs = [(1.0 + xv) * (1.0 + xv) for xv in xs]
            gts = [jnp.where(xv >= hi_fv, sq, 0.0) for xv, sq in zip(xs, sqs)]
            ges = [jnp.where(xv >= lo_fv, sq, 0.0) for xv, sq in zip(xs, sqs)]
            for u in range(0, _U, 2):
                ga = ga + gts[u]
                ea = ea + ges[u]
                gb = gb + gts[u + 1]
                eb = eb + ges[u + 1]
            return ga, gb, ea, eb

        z16 = jnp.zeros((16,), jnp.float32)
        ga, gb, ea, eb = lax.fori_loop(0, _NVEC // _U, st_body,
                                       (z16, z16, z16, z16))
        s_gt = _sscal(ga + gb)
        s_b = _sscal(ea + eb) - s_gt
        ratio = _scal(jnp.full((16,), s_b) / jnp.full((16,), c_b))
        neg = (s_gt + (kf - c_gt) * ratio) * jnp.float32(1.0 / _K)
        l_r = _DELTA * (1.0 - pos) * (1.0 - pos) + neg
        return ls + l_r

    pltpu.make_async_copy(inputs_hbm.at[base], rowbuf.at[0], sem0).start()

    def pair_body(t, ls):
        r0 = 2 * t
        r1 = r0 + 1
        pltpu.make_async_copy(inputs_hbm.at[base + r0], rowbuf.at[0], sem0).wait()
        pltpu.make_async_copy(inputs_hbm.at[base + r1], rowbuf.at[1], sem1).start()
        ls = process_row(0, r0, ls)
        pltpu.make_async_copy(inputs_hbm.at[base + r1], rowbuf.at[1], sem1).wait()
        nxt = jnp.minimum(base + r0 + 2, jnp.int32(_M - 1))
        pltpu.make_async_copy(inputs_hbm.at[nxt], rowbuf.at[0], sem0).start()
        ls = process_row(1, r1, ls)
        return ls

    ls = lax.fori_loop(0, _ROWS_PER // 2, pair_body, jnp.float32(0.0))
    pltpu.make_async_copy(inputs_hbm.at[jnp.minimum(base + _ROWS_PER,
                                                    jnp.int32(_M - 1))],
                          rowbuf.at[0], sem0).wait()
    outv[...] = jnp.where(lane0, ls, 0.0)
    pltpu.sync_copy(outv, out_hbm.at[wid])


def _key_of_tc(x):
    i = jax.lax.bitcast_convert_type(x, jnp.int32)
    return i ^ (lax.shift_right_arithmetic(i, 31) & jnp.int32(0x7FFFFFFF))


def _float_of_key_tc(key):
    i = jnp.where(key >= 0, key, key ^ jnp.int32(0x7FFFFFFF))
    return jax.lax.bitcast_convert_type(i, jnp.float32)


def _tc_block(inputs_ref, tgt_ref, out_ref, *, k, n, rows):
    x = inputs_ref[...]
    tgt = tgt_ref[...]
    col = jax.lax.broadcasted_iota(jnp.int32, (rows, n), 1)
    pos_mask = col == tgt
    pos = jnp.sum(jnp.where(pos_mask, x, 0.0), axis=1, keepdims=True)
    x = jnp.where(pos_mask, jnp.float32(_NEG_FILL), x)

    kf = jnp.float32(k)
    mx = jnp.max(x, axis=1, keepdims=True)
    mn = jnp.min(x, axis=1, keepdims=True)
    lo = _key_of_tc(mn)
    hi = _key_of_tc(mx) + 1

    def body(_, carry):
        lo, hi = carry
        mid = (lo & hi) + lax.shift_right_arithmetic(lo ^ hi, 1)
        t = _float_of_key_tc(mid)
        cnt = jnp.sum(jnp.where(x >= t, 1.0, 0.0), axis=1, keepdims=True)
        take = cnt >= kf
        return jnp.where(take, mid, lo), jnp.where(take, hi, mid)

    lo, hi = jax.lax.fori_loop(0, 18, body, (lo, hi))
    lo_f = _float_of_key_tc(lo)
    hi_f = _float_of_key_tc(hi)
    sq = (1.0 + x) ** 2
    ge_hi = x >= hi_f
    ge_lo = x >= lo_f
    cnt_gt = jnp.sum(jnp.where(ge_hi, 1.0, 0.0), axis=1, keepdims=True)
    s_gt = jnp.sum(jnp.where(ge_hi, sq, 0.0), axis=1, keepdims=True)
    cnt_ge = jnp.sum(jnp.where(ge_lo, 1.0, 0.0), axis=1, keepdims=True)
    s_ge = jnp.sum(jnp.where(ge_lo, sq, 0.0), axis=1, keepdims=True)
    cnt_b = cnt_ge - cnt_gt
    s_b = s_ge - s_gt
    neg = (s_gt + (kf - cnt_gt) * s_b / cnt_b) / kf
    l = _DELTA * (1.0 - pos) ** 2 + neg

    @pl.when(pl.program_id(0) == 0)
    def _():
        out_ref[...] = jnp.zeros_like(out_ref)

    out_ref[...] += jnp.sum(l, axis=0, keepdims=True)


def kernel(inputs, targets_, targets, GT_MC):
    m, n = inputs.shape
    tgt_i32 = targets.astype(jnp.int32)
    mesh = plsc.VectorSubcoreMesh(core_axis_name="c", subcore_axis_name="s")
    sc_fn = functools.partial(
        pl.kernel,
        mesh=mesh,
        compiler_params=pltpu.CompilerParams(needs_layout_passes=False),
        out_type=jax.ShapeDtypeStruct((_NW, 16), jnp.float32),
        scratch_types=[
            pltpu.VMEM((2, _N), jnp.float32),
            pltpu.VMEM((_HPAD,), jnp.float32),
            pltpu.VMEM((_HPAD,), jnp.float32),
            pltpu.VMEM((_TGTBUF,), jnp.int32),
            pltpu.VMEM((16,), jnp.float32),
            pltpu.SemaphoreType.DMA,
            pltpu.SemaphoreType.DMA,
        ],
    )(_sc_body)
    sc_out = sc_fn(inputs, tgt_i32)

    tc_grid = (m - _SROWS) // _TROWS
    off = _SROWS // _TROWS
    tc_out = pl.pallas_call(
        functools.partial(_tc_block, k=_K, n=n, rows=_TROWS),
        grid=(tc_grid,),
        in_specs=[
            pl.BlockSpec((_TROWS, n), lambda i: (i + off, 0)),
            pl.BlockSpec((_TROWS, 1), lambda i: (i + off, 0)),
        ],
        out_specs=pl.BlockSpec((1, 1), lambda i: (0, 0)),
        out_shape=jax.ShapeDtypeStruct((1, 1), jnp.float32),
    )(inputs, tgt_i32.reshape(m, 1))

    return (jnp.sum(sc_out) + tc_out[0, 0]) / m


# speedup vs baseline: 1.2095x; 1.1068x over previous
import functools

import jax
import jax.numpy as jnp
from jax import lax
from jax.experimental import pallas as pl
from jax.experimental.pallas import tpu as pltpu
from jax.experimental.pallas import tpu_sc as plsc

_DELTA = 5.0
_R = 0.01
_NEG_FILL = -1e9

_M, _N = 1024, 16384
_K = int(_R * (_N - 1))
_NW = 32
_SROWS = 448
_ROWS_PER = _SROWS // _NW
_NVEC = _N // 16
_U = 8
_HPAD = 4096
_TROWS = 32
_TGTBUF = 32


def _scal(v):
    return lax.reduce_max(v, axes=(0,)) if getattr(v, "ndim", 0) else v


def _sscal(v):
    return lax.reduce_sum(v, axes=(0,)) if getattr(v, "ndim", 0) else v


def _key_vec(xv):
    iv = plsc.bitcast(xv, jnp.int32)
    return iv ^ (lax.shift_right_arithmetic(iv, 31) & jnp.int32(0x7FFFFFFF))


def _float_of_key_vec(kscal):
    kv = jnp.full((16,), kscal, dtype=jnp.int32)
    iv = jnp.where(kv >= 0, kv, kv ^ jnp.int32(0x7FFFFFFF))
    return plsc.bitcast(iv, jnp.float32)


def _scan_histogram(h_ref, k_rem):
    iota16 = lax.iota(jnp.int32, 16)
    tsum = []
    for c in range(16):
        acc = jnp.zeros((16,), jnp.float32)
        for d in range(16):
            acc = acc + h_ref[pl.ds((c * 16 + d) * 16, 16)]
        tsum.append(_sscal(acc))
    suf = [jnp.float32(0.0)] * 17
    for c in range(15, -1, -1):
        suf[c] = suf[c + 1] + tsum[c]
    suf_v = jnp.zeros((16,), jnp.float32)
    for c in range(16):
        suf_v = jnp.where(iota16 == c, suf[c], suf_v)
    m1 = suf_v >= k_rem
    c_star = _scal(plsc.all_reduce_population_count(m1)) - 1
    above_v = jnp.zeros((16,), jnp.float32)
    for c in range(16):
        above_v = jnp.where(jnp.int32(c) == c_star + 1, suf[c], above_v)
    above = _scal(above_v)
    tc = jnp.zeros((16,), jnp.float32)
    for j in range(16):
        bv = h_ref[pl.ds((c_star * 16 + j) * 16, 16)]
        tc = jnp.where(iota16 == j, _sscal(bv), tc)
    total_c = _sscal(tc)
    ws = total_c - plsc.cumsum(tc) + tc
    m2 = (above + ws) >= k_rem
    j_star = _scal(plsc.all_reduce_population_count(m2)) - 1
    ws_j = _sscal(jnp.where(iota16 == j_star, ws, 0.0))
    cnt_j = _sscal(jnp.where(iota16 == j_star, tc, 0.0))
    bucket = c_star * 16 + j_star
    c_above = above + (ws_j - cnt_j)
    return bucket, c_above, cnt_j


def _sc_body(inputs_hbm, targets_hbm, out_hbm, rowbuf, h1, h2, tgtv, outv,
             sem0, sem1):
    nc = 2
    wid = lax.axis_index("s") * nc + lax.axis_index("c")
    base = wid * _ROWS_PER
    iota16 = lax.iota(jnp.int32, 16)
    ones = jnp.full((16,), 1.0, dtype=jnp.float32)
    lane0 = iota16 == 0
    kf = jnp.float32(_K)

    toff = base & 7
    abase = pl.multiple_of(base - toff, 8)
    pltpu.sync_copy(targets_hbm.at[pl.ds(abase, _TGTBUF)], tgtv)

    def process_row(b, r, ls):
        bvec = jnp.full((16,), b, jnp.int32)
        tvec = plsc.load_gather(tgtv, [jnp.full((16,), toff + r, jnp.int32)])
        pos16 = plsc.load_gather(rowbuf, [bvec, tvec])
        pos = _scal(pos16)
        plsc.store_scatter(rowbuf, [bvec, tvec],
                           jnp.full((16,), _NEG_FILL, jnp.float32), mask=lane0)

        def zero_body(i, _):
            for u in range(_U):
                h1[pl.ds((i * _U + u) * 16, 16)] = jnp.zeros((16,), jnp.float32)
                h2[pl.ds((i * _U + u) * 16, 16)] = jnp.zeros((16,), jnp.float32)
            return 0

        lax.fori_loop(0, _HPAD // 16 // _U, zero_body, 0)

        def l1_body(i, _):
            xs = [rowbuf[b, pl.ds((i * _U + u) * 16, 16)] for u in range(_U)]
            addrs = []
            for u in range(_U):
                key = _key_vec(xs[u])
                d1 = lax.shift_right_arithmetic(key, 24) + 128
                addrs.append(lax.shift_left(d1, 4) + iota16)
            for u in range(_U):
                plsc.addupdate_scatter(h1, [addrs[u]], ones)
            return 0

        lax.fori_loop(0, _NVEC // _U, l1_body, 0)
        b1, c_gt1, _ = _scan_histogram(h1, kf)

        def l2_body(i, _):
            xs = [rowbuf[b, pl.ds((i * _U + u) * 16, 16)] for u in range(_U)]
            addrs, masks = [], []
            for u in range(_U):
                key = _key_vec(xs[u])
                d1 = lax.shift_right_arithmetic(key, 24) + 128
                d2s = lax.shift_right_arithmetic(key, 12) & jnp.int32(0xFF0)
                addrs.append(d2s + iota16)
                masks.append(d1 == b1)
            for u in range(_U):
                plsc.addupdate_scatter(h2, [addrs[u]], ones, mask=masks[u])
            return 0

        lax.fori_loop(0, _NVEC // _U, l2_body, 0)
        b2, c_gt2, c_b = _scan_histogram(h2, kf - c_gt1)
        c_gt = c_gt1 + c_gt2

        lo_key = lax.shift_left(b1 - 128, 24) + lax.shift_left(b2, 16)
        hi_key = lo_key + jnp.int32(1 << 16)
        lo_fv = _float_of_key_vec(lo_key)
        hi_fv = _float_of_key_vec(hi_key)

        def st_body(i, carry):
            ga, gb, ea, eb = carry
            xs = [rowbuf[b, pl.ds((i * _U + u) * 16, 16)] for u in range(_U)]
            sqs = [(1.0 + xv) * (1.0 + xv) for xv in xs]
            gts = [jnp.where(xv >= hi_fv, sq, 0.0) for xv, sq in zip(xs, sqs)]
            ges = [jnp.where(xv >= lo_fv, sq, 0.0) for xv, sq in zip(xs, sqs)]
            for u in range(0, _U, 2):
                ga = ga + gts[u]
                ea = ea + ges[u]
                gb = gb + gts[u + 1]
                eb = eb + ges[u + 1]
            return ga, gb, ea, eb

        z16 = jnp.zeros((16,), jnp.float32)
        ga, gb, ea, eb = lax.fori_loop(0, _NVEC // _U, st_body,
                                       (z16, z16, z16, z16))
        s_gt = _sscal(ga + gb)
        s_b = _sscal(ea + eb) - s_gt
        ratio = _scal(jnp.full((16,), s_b) / jnp.full((16,), c_b))
        neg = (s_gt + (kf - c_gt) * ratio) * jnp.float32(1.0 / _K)
        l_r = _DELTA * (1.0 - pos) * (1.0 - pos) + neg
        return ls + l_r

    pltpu.make_async_copy(inputs_hbm.at[base], rowbuf.at[0], sem0).start()

    def pair_body(t, ls):
        r0 = 2 * t
        r1 = r0 + 1
        pltpu.make_async_copy(inputs_hbm.at[base + r0], rowbuf.at[0], sem0).wait()
        pltpu.make_async_copy(inputs_hbm.at[base + r1], rowbuf.at[1], sem1).start()
        ls = process_row(0, r0, ls)
        pltpu.make_async_copy(inputs_hbm.at[base + r1], rowbuf.at[1], sem1).wait()
        nxt = jnp.minimum(base + r0 + 2, jnp.int32(_M - 1))
        pltpu.make_async_copy(inputs_hbm.at[nxt], rowbuf.at[0], sem0).start()
        ls = process_row(1, r1, ls)
        return ls

    ls = lax.fori_loop(0, _ROWS_PER // 2, pair_body, jnp.float32(0.0))
    pltpu.make_async_copy(inputs_hbm.at[jnp.minimum(base + _ROWS_PER,
                                                    jnp.int32(_M - 1))],
                          rowbuf.at[0], sem0).wait()
    outv[...] = jnp.where(lane0, ls, 0.0)
    pltpu.sync_copy(outv, out_hbm.at[wid])


def _key_of_tc(x):
    i = jax.lax.bitcast_convert_type(x, jnp.int32)
    return i ^ (lax.shift_right_arithmetic(i, 31) & jnp.int32(0x7FFFFFFF))


def _float_of_key_tc(key):
    i = jnp.where(key >= 0, key, key ^ jnp.int32(0x7FFFFFFF))
    return jax.lax.bitcast_convert_type(i, jnp.float32)


def _tc_block(inputs_ref, tgt_ref, out_ref, *, k, n, rows):
    x = inputs_ref[...]
    tgt = tgt_ref[...]
    col = jax.lax.broadcasted_iota(jnp.int32, (rows, n), 1)
    pos_mask = col == tgt
    pos = jnp.sum(jnp.where(pos_mask, x, 0.0), axis=1, keepdims=True)
    x = jnp.where(pos_mask, jnp.float32(_NEG_FILL), x)

    kf = jnp.float32(k)
    mx = jnp.max(x, axis=1, keepdims=True)
    mn = jnp.min(x, axis=1, keepdims=True)
    lo = _key_of_tc(mn)
    hi = _key_of_tc(mx) + 1

    def body(_, carry):
        lo, hi = carry
        mid = (lo & hi) + lax.shift_right_arithmetic(lo ^ hi, 1)
        t = _float_of_key_tc(mid)
        cnt = jnp.sum(jnp.where(x >= t, 1.0, 0.0), axis=1, keepdims=True)
        take = cnt >= kf
        return jnp.where(take, mid, lo), jnp.where(take, hi, mid)

    lo, hi = jax.lax.fori_loop(0, 18, body, (lo, hi))
    lo_f = _float_of_key_tc(lo)
    hi_f = _float_of_key_tc(hi)
    sq = (1.0 + x) ** 2
    ge_hi = x >= hi_f
    ge_lo = x >= lo_f
    cnt_gt = jnp.sum(jnp.where(ge_hi, 1.0, 0.0), axis=1, keepdims=True)
    s_gt = jnp.sum(jnp.where(ge_hi, sq, 0.0), axis=1, keepdims=True)
    cnt_ge = jnp.sum(jnp.where(ge_lo, 1.0, 0.0), axis=1, keepdims=True)
    s_ge = jnp.sum(jnp.where(ge_lo, sq, 0.0), axis=1, keepdims=True)
    cnt_b = cnt_ge - cnt_gt
    s_b = s_ge - s_gt
    neg = (s_gt + (kf - cnt_gt) * s_b / cnt_b) / kf
    l = _DELTA * (1.0 - pos) ** 2 + neg

    @pl.when(pl.program_id(0) == 0)
    def _():
        out_ref[...] = jnp.zeros_like(out_ref)

    out_ref[...] += jnp.sum(l, axis=0, keepdims=True)


def kernel(inputs, targets_, targets, GT_MC):
    m, n = inputs.shape
    tgt_i32 = targets.astype(jnp.int32)
    mesh = plsc.VectorSubcoreMesh(core_axis_name="c", subcore_axis_name="s")
    sc_fn = functools.partial(
        pl.kernel,
        mesh=mesh,
        compiler_params=pltpu.CompilerParams(needs_layout_passes=False),
        out_type=jax.ShapeDtypeStruct((_NW, 16), jnp.float32),
        scratch_types=[
            pltpu.VMEM((2, _N), jnp.float32),
            pltpu.VMEM((_HPAD,), jnp.float32),
            pltpu.VMEM((_HPAD,), jnp.float32),
            pltpu.VMEM((_TGTBUF,), jnp.int32),
            pltpu.VMEM((16,), jnp.float32),
            pltpu.SemaphoreType.DMA,
            pltpu.SemaphoreType.DMA,
        ],
    )(_sc_body)
    sc_out = sc_fn(inputs, tgt_i32)

    tc_grid = (m - _SROWS) // _TROWS
    off = _SROWS // _TROWS
    tc_out = pl.pallas_call(
        functools.partial(_tc_block, k=_K, n=n, rows=_TROWS),
        grid=(tc_grid,),
        in_specs=[
            pl.BlockSpec((_TROWS, n), lambda i: (i + off, 0)),
            pl.BlockSpec((_TROWS, 1), lambda i: (i + off, 0)),
        ],
        out_specs=pl.BlockSpec((1, 1), lambda i: (0, 0)),
        out_shape=jax.ShapeDtypeStruct((1, 1), jnp.float32),
    )(inputs, tgt_i32.reshape(m, 1))

    return (jnp.sum(sc_out) + tc_out[0, 0]) / m


# speedup vs baseline: 1.3403x; 1.1082x over previous
import functools

import jax
import jax.numpy as jnp
from jax import lax
from jax.experimental import pallas as pl
from jax.experimental.pallas import tpu as pltpu
from jax.experimental.pallas import tpu_sc as plsc

_DELTA = 5.0
_R = 0.01
_NEG_FILL = -1e9

_M, _N = 1024, 16384
_K = int(_R * (_N - 1))
_NW = 32
_SROWS = 448
_ROWS_PER = _SROWS // _NW
_NVEC = _N // 16
_U = 8
_HPAD = 4096
_TROWS = 64
_TGTBUF = 32


def _scal(v):
    return lax.reduce_max(v, axes=(0,)) if getattr(v, "ndim", 0) else v


def _sscal(v):
    return lax.reduce_sum(v, axes=(0,)) if getattr(v, "ndim", 0) else v


def _key_vec(xv):
    iv = plsc.bitcast(xv, jnp.int32)
    return iv ^ (lax.shift_right_arithmetic(iv, 31) & jnp.int32(0x7FFFFFFF))


def _float_of_key_vec(kscal):
    kv = jnp.full((16,), kscal, dtype=jnp.int32)
    iv = jnp.where(kv >= 0, kv, kv ^ jnp.int32(0x7FFFFFFF))
    return plsc.bitcast(iv, jnp.float32)


def _scan_histogram(h_ref, k_rem):
    iota16 = lax.iota(jnp.int32, 16)
    tsum = []
    for c in range(16):
        acc = jnp.zeros((16,), jnp.float32)
        for d in range(16):
            acc = acc + h_ref[pl.ds((c * 16 + d) * 16, 16)]
        tsum.append(_sscal(acc))
    suf = [jnp.float32(0.0)] * 17
    for c in range(15, -1, -1):
        suf[c] = suf[c + 1] + tsum[c]
    suf_v = jnp.zeros((16,), jnp.float32)
    for c in range(16):
        suf_v = jnp.where(iota16 == c, suf[c], suf_v)
    m1 = suf_v >= k_rem
    c_star = _scal(plsc.all_reduce_population_count(m1)) - 1
    above_v = jnp.zeros((16,), jnp.float32)
    for c in range(16):
        above_v = jnp.where(jnp.int32(c) == c_star + 1, suf[c], above_v)
    above = _scal(above_v)
    tc = jnp.zeros((16,), jnp.float32)
    for j in range(16):
        bv = h_ref[pl.ds((c_star * 16 + j) * 16, 16)]
        tc = jnp.where(iota16 == j, _sscal(bv), tc)
    total_c = _sscal(tc)
    ws = total_c - plsc.cumsum(tc) + tc
    m2 = (above + ws) >= k_rem
    j_star = _scal(plsc.all_reduce_population_count(m2)) - 1
    ws_j = _sscal(jnp.where(iota16 == j_star, ws, 0.0))
    cnt_j = _sscal(jnp.where(iota16 == j_star, tc, 0.0))
    bucket = c_star * 16 + j_star
    c_above = above + (ws_j - cnt_j)
    return bucket, c_above, cnt_j


def _sc_body(inputs_hbm, targets_hbm, out_hbm, rowbuf, h1, h2, tgtv, outv,
             sem0, sem1):
    nc = 2
    wid = lax.axis_index("s") * nc + lax.axis_index("c")
    base = wid * _ROWS_PER
    iota16 = lax.iota(jnp.int32, 16)
    ones = jnp.full((16,), 1.0, dtype=jnp.float32)
    lane0 = iota16 == 0
    kf = jnp.float32(_K)

    toff = base & 7
    abase = pl.multiple_of(base - toff, 8)
    pltpu.sync_copy(targets_hbm.at[pl.ds(abase, _TGTBUF)], tgtv)

    def process_row(b, r, ls):
        bvec = jnp.full((16,), b, jnp.int32)
        tvec = plsc.load_gather(tgtv, [jnp.full((16,), toff + r, jnp.int32)])
        pos16 = plsc.load_gather(rowbuf, [bvec, tvec])
        pos = _scal(pos16)
        plsc.store_scatter(rowbuf, [bvec, tvec],
                           jnp.full((16,), _NEG_FILL, jnp.float32), mask=lane0)

        def zero_body(i, _):
            for u in range(_U):
                h1[pl.ds((i * _U + u) * 16, 16)] = jnp.zeros((16,), jnp.float32)
                h2[pl.ds((i * _U + u) * 16, 16)] = jnp.zeros((16,), jnp.float32)
            return 0

        lax.fori_loop(0, _HPAD // 16 // _U, zero_body, 0)

        def l1_body(i, _):
            xs = [rowbuf[b, pl.ds((i * _U + u) * 16, 16)] for u in range(_U)]
            addrs = []
            for u in range(_U):
                key = _key_vec(xs[u])
                d1 = lax.shift_right_arithmetic(key, 24) + 128
                addrs.append(lax.shift_left(d1, 4) + iota16)
            for u in range(_U):
                plsc.addupdate_scatter(h1, [addrs[u]], ones)
            return 0

        lax.fori_loop(0, _NVEC // _U, l1_body, 0)
        b1, c_gt1, _ = _scan_histogram(h1, kf)

        def l2_body(i, _):
            xs = [rowbuf[b, pl.ds((i * _U + u) * 16, 16)] for u in range(_U)]
            addrs, masks = [], []
            for u in range(_U):
                key = _key_vec(xs[u])
                d1 = lax.shift_right_arithmetic(key, 24) + 128
                d2s = lax.shift_right_arithmetic(key, 12) & jnp.int32(0xFF0)
                addrs.append(d2s + iota16)
                masks.append(d1 == b1)
            for u in range(_U):
                plsc.addupdate_scatter(h2, [addrs[u]], ones, mask=masks[u])
            return 0

        lax.fori_loop(0, _NVEC // _U, l2_body, 0)
        b2, c_gt2, c_b = _scan_histogram(h2, kf - c_gt1)
        c_gt = c_gt1 + c_gt2

        lo_key = lax.shift_left(b1 - 128, 24) + lax.shift_left(b2, 16)
        hi_key = lo_key + jnp.int32(1 << 16)
        lo_fv = _float_of_key_vec(lo_key)
        hi_fv = _float_of_key_vec(hi_key)

        def st_body(i, carry):
            ga, gb, ea, eb = carry
            xs = [rowbuf[b, pl.ds((i * _U + u) * 16, 16)] for u in range(_U)]
            sqs = [(1.0 + xv) * (1.0 + xv) for xv in xs]
            gts = [jnp.where(xv >= hi_fv, sq, 0.0) for xv, sq in zip(xs, sqs)]
            ges = [jnp.where(xv >= lo_fv, sq, 0.0) for xv, sq in zip(xs, sqs)]
            for u in range(0, _U, 2):
                ga = ga + gts[u]
                ea = ea + ges[u]
                gb = gb + gts[u + 1]
                eb = eb + ges[u + 1]
            return ga, gb, ea, eb

        z16 = jnp.zeros((16,), jnp.float32)
        ga, gb, ea, eb = lax.fori_loop(0, _NVEC // _U, st_body,
                                       (z16, z16, z16, z16))
        s_gt = _sscal(ga + gb)
        s_b = _sscal(ea + eb) - s_gt
        ratio = _scal(jnp.full((16,), s_b) / jnp.full((16,), c_b))
        neg = (s_gt + (kf - c_gt) * ratio) * jnp.float32(1.0 / _K)
        l_r = _DELTA * (1.0 - pos) * (1.0 - pos) + neg
        return ls + l_r

    pltpu.make_async_copy(inputs_hbm.at[base], rowbuf.at[0], sem0).start()

    def pair_body(t, ls):
        r0 = 2 * t
        r1 = r0 + 1
        pltpu.make_async_copy(inputs_hbm.at[base + r0], rowbuf.at[0], sem0).wait()
        pltpu.make_async_copy(inputs_hbm.at[base + r1], rowbuf.at[1], sem1).start()
        ls = process_row(0, r0, ls)
        pltpu.make_async_copy(inputs_hbm.at[base + r1], rowbuf.at[1], sem1).wait()
        nxt = jnp.minimum(base + r0 + 2, jnp.int32(_M - 1))
        pltpu.make_async_copy(inputs_hbm.at[nxt], rowbuf.at[0], sem0).start()
        ls = process_row(1, r1, ls)
        return ls

    ls = lax.fori_loop(0, _ROWS_PER // 2, pair_body, jnp.float32(0.0))
    pltpu.make_async_copy(inputs_hbm.at[jnp.minimum(base + _ROWS_PER,
                                                    jnp.int32(_M - 1))],
                          rowbuf.at[0], sem0).wait()
    outv[...] = jnp.where(lane0, ls, 0.0)
    pltpu.sync_copy(outv, out_hbm.at[wid])


def _key_of_tc(x):
    i = jax.lax.bitcast_convert_type(x, jnp.int32)
    return i ^ (lax.shift_right_arithmetic(i, 31) & jnp.int32(0x7FFFFFFF))


def _float_of_key_tc(key):
    i = jnp.where(key >= 0, key, key ^ jnp.int32(0x7FFFFFFF))
    return jax.lax.bitcast_convert_type(i, jnp.float32)


def _tc_block(inputs_ref, tgt_ref, out_ref, *, k, n, rows):
    x = inputs_ref[...]
    tgt = tgt_ref[...]
    col = jax.lax.broadcasted_iota(jnp.int32, (rows, n), 1)
    pos_mask = col == tgt
    pos = jnp.sum(jnp.where(pos_mask, x, 0.0), axis=1, keepdims=True)
    x = jnp.where(pos_mask, jnp.float32(_NEG_FILL), x)

    kf = jnp.float32(k)
    mx = jnp.max(x, axis=1, keepdims=True)
    mn = jnp.min(x, axis=1, keepdims=True)
    lo = _key_of_tc(mn)
    hi = _key_of_tc(mx) + 1

    def body(_, carry):
        lo, hi = carry
        mid = (lo & hi) + lax.shift_right_arithmetic(lo ^ hi, 1)
        t = _float_of_key_tc(mid)
        cnt = jnp.sum(jnp.where(x >= t, 1.0, 0.0), axis=1, keepdims=True)
        take = cnt >= kf
        return jnp.where(take, mid, lo), jnp.where(take, hi, mid)

    lo, hi = jax.lax.fori_loop(0, 18, body, (lo, hi))
    lo_f = _float_of_key_tc(lo)
    hi_f = _float_of_key_tc(hi)
    sq = (1.0 + x) ** 2
    ge_hi = x >= hi_f
    ge_lo = x >= lo_f
    cnt_gt = jnp.sum(jnp.where(ge_hi, 1.0, 0.0), axis=1, keepdims=True)
    s_gt = jnp.sum(jnp.where(ge_hi, sq, 0.0), axis=1, keepdims=True)
    cnt_ge = jnp.sum(jnp.where(ge_lo, 1.0, 0.0), axis=1, keepdims=True)
    s_ge = jnp.sum(jnp.where(ge_lo, sq, 0.0), axis=1, keepdims=True)
    cnt_b = cnt_ge - cnt_gt
    s_b = s_ge - s_gt
    neg = (s_gt + (kf - cnt_gt) * s_b / cnt_b) / kf
    l = _DELTA * (1.0 - pos) ** 2 + neg

    @pl.when(pl.program_id(0) == 0)
    def _():
        out_ref[...] = jnp.zeros_like(out_ref)

    out_ref[...] += jnp.sum(l, axis=0, keepdims=True)


def kernel(inputs, targets_, targets, GT_MC):
    m, n = inputs.shape
    tgt_i32 = targets.astype(jnp.int32)
    mesh = plsc.VectorSubcoreMesh(core_axis_name="c", subcore_axis_name="s")
    sc_fn = functools.partial(
        pl.kernel,
        mesh=mesh,
        compiler_params=pltpu.CompilerParams(needs_layout_passes=False),
        out_type=jax.ShapeDtypeStruct((_NW, 16), jnp.float32),
        scratch_types=[
            pltpu.VMEM((2, _N), jnp.float32),
            pltpu.VMEM((_HPAD,), jnp.float32),
            pltpu.VMEM((_HPAD,), jnp.float32),
            pltpu.VMEM((_TGTBUF,), jnp.int32),
            pltpu.VMEM((16,), jnp.float32),
            pltpu.SemaphoreType.DMA,
            pltpu.SemaphoreType.DMA,
        ],
    )(_sc_body)
    sc_out = sc_fn(inputs, tgt_i32)

    tc_grid = (m - _SROWS) // _TROWS
    off = _SROWS // _TROWS
    tc_out = pl.pallas_call(
        functools.partial(_tc_block, k=_K, n=n, rows=_TROWS),
        grid=(tc_grid,),
        in_specs=[
            pl.BlockSpec((_TROWS, n), lambda i: (i + off, 0)),
            pl.BlockSpec((_TROWS, 1), lambda i: (i + off, 0)),
        ],
        out_specs=pl.BlockSpec((1, 1), lambda i: (0, 0)),
        out_shape=jax.ShapeDtypeStruct((1, 1), jnp.float32),
    )(inputs, tgt_i32.reshape(m, 1))

    return (jnp.sum(sc_out) + tc_out[0, 0]) / m
